# baseline (speedup 1.0000x reference)
import jax
import jax.numpy as jnp
from jax.experimental import pallas as pl
from jax.experimental.pallas import tpu as pltpu

_CROWS = 512
_DEPTH = 8


def _body(eid_ref, a_ref, b_ref, vn_hbm, vo_hbm, h_hbm, out_hbm,
          vn_row, vo_row, ev, in_buf, out_buf, gsem, in_sems, out_sems):
    n = h_hbm.shape[0]
    nchunk = n // _CROWS

    def in_copy(c):
        return pltpu.make_async_copy(
            h_hbm.at[pl.ds(c * _CROWS, _CROWS), :],
            in_buf.at[c % _DEPTH], in_sems.at[c % _DEPTH])

    def out_copy(c):
        return pltpu.make_async_copy(
            out_buf.at[c % _DEPTH],
            out_hbm.at[pl.ds(c * _CROWS, _CROWS), :], out_sems.at[c % _DEPTH])

    eid = eid_ref[0]
    g0 = pltpu.make_async_copy(vn_hbm.at[pl.ds(eid, 1), :], vn_row, gsem.at[0])
    g1 = pltpu.make_async_copy(vo_hbm.at[pl.ds(eid, 1), :], vo_row, gsem.at[1])
    g0.start(); g1.start()
    for c in range(min(_DEPTH, nchunk)):
        in_copy(c).start()
    g0.wait(); g1.wait()
    ev[...] = a_ref[eid] * vn_row[...] + b_ref[eid] * vo_row[...]

    for c in range(nchunk):
        s = c % _DEPTH
        in_copy(c).wait()
        if c >= _DEPTH:
            out_copy(c - _DEPTH).wait()
        out_buf[s] = in_buf[s] + ev[...]
        out_copy(c).start()
        if c + _DEPTH < nchunk:
            in_copy(c + _DEPTH).start()

    for c in range(max(0, nchunk - _DEPTH), nchunk):
        out_copy(c).wait()


def kernel(edit_id, hidden_states, v_new, v_old, alpha, beta):
    B, S, H = hidden_states.shape
    n = B * S
    h2 = hidden_states.reshape(n, H)
    eid = jnp.asarray(edit_id, jnp.int32).reshape(1)
    out = pl.pallas_call(
        _body,
        grid_spec=pltpu.PrefetchScalarGridSpec(
            num_scalar_prefetch=3,
            grid=(1,),
            in_specs=[
                pl.BlockSpec(memory_space=pl.ANY),
                pl.BlockSpec(memory_space=pl.ANY),
                pl.BlockSpec(memory_space=pl.ANY),
            ],
            out_specs=pl.BlockSpec(memory_space=pl.ANY),
            scratch_shapes=[
                pltpu.VMEM((1, H), jnp.float32),
                pltpu.VMEM((1, H), jnp.float32),
                pltpu.VMEM((1, H), jnp.float32),
                pltpu.VMEM((_DEPTH, _CROWS, H), jnp.float32),
                pltpu.VMEM((_DEPTH, _CROWS, H), jnp.float32),
                pltpu.SemaphoreType.DMA((2,)),
                pltpu.SemaphoreType.DMA((_DEPTH,)),
                pltpu.SemaphoreType.DMA((_DEPTH,)),
            ],
        ),
        out_shape=jax.ShapeDtypeStruct((n, H), hidden_states.dtype),
    )(eid, alpha, beta, v_new, v_old, h2)
    return out.reshape(B, S, H)


# speedup vs baseline: 1.0130x; 1.0130x over previous
import jax
import jax.numpy as jnp
from jax.experimental import pallas as pl
from jax.experimental.pallas import tpu as pltpu

_CROWS = 1024
_DEPTH = 6


def _body(eid_ref, a_ref, b_ref, vn_hbm, vo_hbm, h_hbm, out_hbm,
          vn_row, vo_row, ev, in_buf, out_buf, gsem, in_sems, out_sems):
    n = h_hbm.shape[0]
    nchunk = n // _CROWS

    def in_copy(c):
        return pltpu.make_async_copy(
            h_hbm.at[pl.ds(c * _CROWS, _CROWS), :],
            in_buf.at[c % _DEPTH], in_sems.at[c % _DEPTH])

    def out_copy(c):
        return pltpu.make_async_copy(
            out_buf.at[c % _DEPTH],
            out_hbm.at[pl.ds(c * _CROWS, _CROWS), :], out_sems.at[c % _DEPTH])

    eid = eid_ref[0]
    g0 = pltpu.make_async_copy(vn_hbm.at[pl.ds(eid, 1), :], vn_row, gsem.at[0])
    g1 = pltpu.make_async_copy(vo_hbm.at[pl.ds(eid, 1), :], vo_row, gsem.at[1])
    g0.start(); g1.start()
    for c in range(min(_DEPTH, nchunk)):
        in_copy(c).start()
    g0.wait(); g1.wait()
    ev[...] = a_ref[eid] * vn_row[...] + b_ref[eid] * vo_row[...]

    for c in range(nchunk):
        s = c % _DEPTH
        in_copy(c).wait()
        if c >= _DEPTH:
            out_copy(c - _DEPTH).wait()
        out_buf[s] = in_buf[s] + ev[...]
        out_copy(c).start()
        if c + _DEPTH < nchunk:
            in_copy(c + _DEPTH).start()

    for c in range(max(0, nchunk - _DEPTH), nchunk):
        out_copy(c).wait()


def kernel(edit_id, hidden_states, v_new, v_old, alpha, beta):
    B, S, H = hidden_states.shape
    n = B * S
    h2 = hidden_states.reshape(n, H)
    eid = jnp.asarray(edit_id, jnp.int32).reshape(1)
    out = pl.pallas_call(
        _body,
        grid_spec=pltpu.PrefetchScalarGridSpec(
            num_scalar_prefetch=3,
            grid=(1,),
            in_specs=[
                pl.BlockSpec(memory_space=pl.ANY),
                pl.BlockSpec(memory_space=pl.ANY),
                pl.BlockSpec(memory_space=pl.ANY),
            ],
            out_specs=pl.BlockSpec(memory_space=pl.ANY),
            scratch_shapes=[
                pltpu.VMEM((1, H), jnp.float32),
                pltpu.VMEM((1, H), jnp.float32),
                pltpu.VMEM((1, H), jnp.float32),
                pltpu.VMEM((_DEPTH, _CROWS, H), jnp.float32),
                pltpu.VMEM((_DEPTH, _CROWS, H), jnp.float32),
                pltpu.SemaphoreType.DMA((2,)),
                pltpu.SemaphoreType.DMA((_DEPTH,)),
                pltpu.SemaphoreType.DMA((_DEPTH,)),
            ],
        ),
        out_shape=jax.ShapeDtypeStruct((n, H), hidden_states.dtype),
    )(eid, alpha, beta, v_new, v_old, h2)
    return out.reshape(B, S, H)
